# baseline (speedup 1.0000x reference)
import jax
import jax.numpy as jnp
from jax.experimental import pallas as pl
from jax.experimental.pallas import tpu as pltpu

_R = 256
_BJ = 1024


def _cn_kernel(thr_ref, a_ref, label_ref, mask_ref, out_ref,
               lb_ref, acc_c_ref, acc_n_ref):
    j = pl.program_id(0)
    nj = pl.num_programs(0)
    R = lb_ref.shape[0]

    @pl.when(j == 0)
    def _init():
        lb_ref[...] = (a_ref[:R, :] != 0.0).astype(jnp.bfloat16)

    ab = (a_ref[...] != 0.0).astype(jnp.bfloat16)
    s = jax.lax.dot_general(
        lb_ref[...], ab, (((1,), (1,)), ((), ())),
        preferred_element_type=jnp.float32)

    pred = s >= thr_ref[0]
    label = label_ref[...] != 0.0
    m = mask_ref[...] != 0.0
    correct = jnp.where(m & (pred == label), 1.0, 0.0)
    mf = jnp.where(m, 1.0, 0.0)

    @pl.when(j == 0)
    def _first():
        acc_c_ref[...] = correct
        acc_n_ref[...] = mf

    @pl.when(j > 0)
    def _accum():
        acc_c_ref[...] += correct
        acc_n_ref[...] += mf

    @pl.when(j == nj - 1)
    def _fin():
        out_ref[0, 0] = jnp.sum(acc_c_ref[...]) / jnp.sum(acc_n_ref[...])


@jax.jit
def kernel(A_train, A_full, mask, best_threshold):
    N = A_train.shape[0]
    R, C = mask.shape
    nj = C // _BJ

    thr = jnp.reshape(best_threshold.astype(jnp.float32), (1,))

    out = pl.pallas_call(
        _cn_kernel,
        grid=(nj,),
        in_specs=[
            pl.BlockSpec(memory_space=pltpu.SMEM),
            pl.BlockSpec((_BJ, N), lambda j: (j, 0)),
            pl.BlockSpec((R, _BJ), lambda j: (0, j)),
            pl.BlockSpec((R, _BJ), lambda j: (0, j)),
        ],
        out_specs=pl.BlockSpec(memory_space=pltpu.SMEM),
        out_shape=jax.ShapeDtypeStruct((1, 1), jnp.float32),
        scratch_shapes=[
            pltpu.VMEM((R, N), jnp.bfloat16),
            pltpu.VMEM((R, _BJ), jnp.float32),
            pltpu.VMEM((R, _BJ), jnp.float32),
        ],
    )(thr, A_train, A_full, mask)
    return out[0, 0]


# speedup vs baseline: 1.0655x; 1.0655x over previous
import jax
import jax.numpy as jnp
from jax.experimental import pallas as pl
from jax.experimental.pallas import tpu as pltpu

_R = 256
_BJ = 512


def _cn_kernel(thr_ref, a_ref, label_ref, mask_ref, out_ref,
               lb_ref, acc_c_ref, acc_n_ref):
    j = pl.program_id(0)
    nj = pl.num_programs(0)
    R = lb_ref.shape[0]

    @pl.when(j == 0)
    def _init():
        lb_ref[...] = a_ref[:R, :].astype(jnp.bfloat16)

    ab = a_ref[...].astype(jnp.bfloat16)
    s = jax.lax.dot_general(
        lb_ref[...], ab, (((1,), (1,)), ((), ())),
        preferred_element_type=jnp.float32)

    pred = s >= thr_ref[0]
    label = label_ref[...] != 0.0
    m = mask_ref[...] != 0.0
    correct = jnp.where(m & (pred == label), 1.0, 0.0)
    mf = jnp.where(m, 1.0, 0.0)

    @pl.when(j == 0)
    def _first():
        acc_c_ref[...] = correct
        acc_n_ref[...] = mf

    @pl.when(j > 0)
    def _accum():
        acc_c_ref[...] += correct
        acc_n_ref[...] += mf

    @pl.when(j == nj - 1)
    def _fin():
        out_ref[0, 0] = jnp.sum(acc_c_ref[...]) / jnp.sum(acc_n_ref[...])


@jax.jit
def kernel(A_train, A_full, mask, best_threshold):
    N = A_train.shape[0]
    R, C = mask.shape
    nj = C // _BJ

    thr = jnp.reshape(best_threshold.astype(jnp.float32), (1,))

    out = pl.pallas_call(
        _cn_kernel,
        grid=(nj,),
        in_specs=[
            pl.BlockSpec(memory_space=pltpu.SMEM),
            pl.BlockSpec((_BJ, N), lambda j: (j, 0)),
            pl.BlockSpec((R, _BJ), lambda j: (0, j)),
            pl.BlockSpec((R, _BJ), lambda j: (0, j)),
        ],
        out_specs=pl.BlockSpec(memory_space=pltpu.SMEM),
        out_shape=jax.ShapeDtypeStruct((1, 1), jnp.float32),
        scratch_shapes=[
            pltpu.VMEM((R, N), jnp.bfloat16),
            pltpu.VMEM((R, _BJ), jnp.float32),
            pltpu.VMEM((R, _BJ), jnp.float32),
        ],
    )(thr, A_train, A_full, mask)
    return out[0, 0]


# speedup vs baseline: 1.0832x; 1.0167x over previous
import jax
import jax.numpy as jnp
from jax.experimental import pallas as pl
from jax.experimental.pallas import tpu as pltpu

_R = 256
_BJ = 512


def _cn_kernel(thr_ref, a_ref, label_ref, mask_ref, out_ref,
               lb_ref, acc_c_ref, acc_n_ref):
    j = pl.program_id(0)
    nj = pl.num_programs(0)
    R = lb_ref.shape[0]

    @pl.when(j == 0)
    def _init():
        lb_ref[...] = (a_ref[:R, :] != 0.0).astype(jnp.bfloat16)

    ab = (a_ref[...] != 0.0).astype(jnp.bfloat16)
    s = jax.lax.dot_general(
        lb_ref[...], ab, (((1,), (1,)), ((), ())),
        preferred_element_type=jnp.float32)

    pred = s >= thr_ref[0]
    label = label_ref[...] != 0.0
    m = mask_ref[...] != 0.0
    correct = jnp.where(m & (pred == label), 1.0, 0.0)
    mf = jnp.where(m, 1.0, 0.0)

    @pl.when(j == 0)
    def _first():
        acc_c_ref[...] = correct
        acc_n_ref[...] = mf

    @pl.when(j > 0)
    def _accum():
        acc_c_ref[...] += correct
        acc_n_ref[...] += mf

    @pl.when(j == nj - 1)
    def _fin():
        out_ref[0, 0] = jnp.sum(acc_c_ref[...]) / jnp.sum(acc_n_ref[...])


@jax.jit
def kernel(A_train, A_full, mask, best_threshold):
    N = A_train.shape[0]
    R, C = mask.shape
    nj = C // _BJ

    thr = jnp.reshape(best_threshold.astype(jnp.float32), (1,))

    out = pl.pallas_call(
        _cn_kernel,
        grid=(nj,),
        in_specs=[
            pl.BlockSpec(memory_space=pltpu.SMEM),
            pl.BlockSpec((_BJ, N), lambda j: (j, 0)),
            pl.BlockSpec((R, _BJ), lambda j: (0, j)),
            pl.BlockSpec((R, _BJ), lambda j: (0, j)),
        ],
        out_specs=pl.BlockSpec(memory_space=pltpu.SMEM),
        out_shape=jax.ShapeDtypeStruct((1, 1), jnp.float32),
        scratch_shapes=[
            pltpu.VMEM((R, N), jnp.bfloat16),
            pltpu.VMEM((R, _BJ), jnp.float32),
            pltpu.VMEM((R, _BJ), jnp.float32),
        ],
    )(thr, A_train, A_full, mask)
    return out[0, 0]
